# baseline (speedup 1.0000x reference)
import functools

import jax
import jax.numpy as jnp
import numpy as np
from jax import lax
from jax.experimental import pallas as pl
from jax.experimental.pallas import tpu as pltpu
from jax.experimental.pallas import tpu_sc as plsc

DIM = 64
HALF = 32
CUTOFF = 5.0
N_CENTERS = 50
KPAD = 64
N_CONV = 3
HW = N_CONV * DIM
EBLK = 1000
NBLK = 10000

NNODE = 50000
NPAD = 50048
NEDGE = 800000
SC_B = 200
SC_PT = NEDGE // (16 * SC_B)
NT = NPAD // 16


def _softplus(x, beta=0.5, threshold=14.0):
    return jnp.where(beta * x > threshold, x,
                     (1.0 / beta) * jnp.log1p(jnp.exp(jnp.minimum(beta * x, threshold))))


def _h_body(d_ref, f_ref, em_ref, w1_ref, b1_ref, w2_ref, b2_ref,
            o0_ref, o1_ref, o2_ref):
    gap = CUTOFF / (N_CENTERS - 1)
    inv_gap = 1.0 / gap
    d = d_ref[...]
    ki = jax.lax.broadcasted_iota(jnp.int32, (1, KPAD), 1)
    centers = jnp.where(ki < N_CENTERS, ki.astype(jnp.float32) * gap, 0.0)
    rbf = jnp.exp(-inv_gap * (d - centers) ** 2)
    w1 = w1_ref[...]
    b1 = b1_ref[...]
    w2 = w2_ref[...]
    b2 = b2_ref[...]
    z = jnp.dot(rbf.astype(jnp.bfloat16), w1, preferred_element_type=jnp.float32) + b1
    h = jnp.dot(_softplus(z).astype(jnp.bfloat16), w2, preferred_element_type=jnp.float32) + b2
    zm = jnp.dot(em_ref[...].astype(jnp.bfloat16), w1, preferred_element_type=jnp.float32) + b1
    hm = jnp.dot(_softplus(zm).astype(jnp.bfloat16), w2, preferred_element_type=jnp.float32) + b2
    h = jnp.where(f_ref[...] > 0.0, hm, h)
    for li, o_ref in enumerate((o0_ref, o1_ref, o2_ref)):
        hl = h[:, li * DIM:(li + 1) * DIM]
        o_ref[...] = jnp.stack([hl[:, :HALF], hl[:, HALF:]], axis=0)


def _compute_h_all(distance, flag, edge_mask_p, W1, B1, W2, B2):
    E = distance.shape[0]
    grid = (E // EBLK,)
    out3 = jax.ShapeDtypeStruct((2, E, HALF), jnp.float32)
    return pl.pallas_call(
        _h_body,
        grid=grid,
        in_specs=[
            pl.BlockSpec((EBLK, 1), lambda i: (i, 0)),
            pl.BlockSpec((EBLK, 1), lambda i: (i, 0)),
            pl.BlockSpec((1, KPAD), lambda i: (0, 0)),
            pl.BlockSpec((KPAD, HW), lambda i: (0, 0)),
            pl.BlockSpec((1, HW), lambda i: (0, 0)),
            pl.BlockSpec((HW, HW), lambda i: (0, 0)),
            pl.BlockSpec((1, HW), lambda i: (0, 0)),
        ],
        out_specs=[pl.BlockSpec((2, EBLK, HALF), lambda i: (0, i, 0))] * 3,
        out_shape=[out3] * 3,
    )(distance[:, None], flag, edge_mask_p,
      W1.astype(jnp.bfloat16), B1, W2.astype(jnp.bfloat16), B2)


def _nn_body(node_ref, w_ref, out_ref):
    nn = jnp.dot(node_ref[...], w_ref[...], preferred_element_type=jnp.float32)
    out_ref[...] = jnp.stack([nn[:, :HALF], nn[:, HALF:]], axis=0)


def _new_node(node, W_nl1):
    N = node.shape[0]
    return pl.pallas_call(
        _nn_body,
        grid=(N // NBLK,),
        in_specs=[
            pl.BlockSpec((NBLK, DIM), lambda i: (i, 0)),
            pl.BlockSpec((DIM, DIM), lambda i: (0, 0)),
        ],
        out_specs=pl.BlockSpec((2, NBLK, HALF), lambda i: (0, i, 0)),
        out_shape=jax.ShapeDtypeStruct((2, N, HALF), jnp.float32),
    )(node, W_nl1)


def _upd_body(node_ref, agg_ref, w2_ref, b2_ref, w3_ref, b3_ref, out_ref):
    agg = jnp.concatenate([agg_ref[0], agg_ref[1]], axis=1)
    cf1 = _softplus(jnp.dot(agg, w2_ref[...], preferred_element_type=jnp.float32) + b2_ref[...])
    out_ref[...] = node_ref[...] + (
        jnp.dot(cf1, w3_ref[...], preferred_element_type=jnp.float32) + b3_ref[...])


def _update_node(node, agg3d, W_nl2, b_nl2, W_nl3, b_nl3):
    N = node.shape[0]
    return pl.pallas_call(
        _upd_body,
        grid=(N // NBLK,),
        in_specs=[
            pl.BlockSpec((NBLK, DIM), lambda i: (i, 0)),
            pl.BlockSpec((2, NBLK, HALF), lambda i: (0, i, 0)),
            pl.BlockSpec((DIM, DIM), lambda i: (0, 0)),
            pl.BlockSpec((1, DIM), lambda i: (0, 0)),
            pl.BlockSpec((DIM, DIM), lambda i: (0, 0)),
            pl.BlockSpec((1, DIM), lambda i: (0, 0)),
        ],
        out_specs=pl.BlockSpec((NBLK, DIM), lambda i: (i, 0)),
        out_shape=jax.ShapeDtypeStruct((N, DIM), jnp.float32),
    )(node, agg3d, W_nl2, b_nl2[None, :], W_nl3, b_nl3[None, :])


def _seg_body(src2_hbm, dst_hbm, nn_hbm, h_hbm, z_hbm, out_hbm, agg_sp,
              idx0, dst0, rows0, h0, idx1, dst1, rows1, h1,
              sl0, sg0, ss0, sl1, sg1, ss1):
    c = lax.axis_index("c")
    s = lax.axis_index("s")
    pltpu.sync_copy(z_hbm.at[pl.ds(s * NT, NT)], agg_sp.at[pl.ds(s * NT, NT)])
    plsc.subcore_barrier()
    cE = c * NEDGE
    base = s * (SC_PT * SC_B)
    bufs = ((idx0, dst0, rows0, h0, sl0, sg0, ss0),
            (idx1, dst1, rows1, h1, sl1, sg1, ss1))

    def issue_loads(t, b):
        idx_v, dst_v, _, h_v, sl, _, _ = bufs[b]
        off = base + t * SC_B
        pltpu.async_copy(src2_hbm.at[pl.ds(cE + off, SC_B)], idx_v, sl)
        pltpu.async_copy(dst_hbm.at[pl.ds(off, SC_B)], dst_v, sl)
        pltpu.async_copy(h_hbm.at[pl.ds(cE + off, SC_B)], h_v, sl)

    def wait_loads(b):
        idx_v, dst_v, _, h_v, sl, _, _ = bufs[b]
        pltpu.make_async_copy(src2_hbm.at[pl.ds(0, SC_B)], idx_v, sl).wait()
        pltpu.make_async_copy(dst_hbm.at[pl.ds(0, SC_B)], dst_v, sl).wait()
        pltpu.make_async_copy(h_hbm.at[pl.ds(0, SC_B)], h_v, sl).wait()

    def issue_gather(b):
        idx_v, _, rows_v, _, _, sg, _ = bufs[b]
        pltpu.async_copy(nn_hbm.at[idx_v], rows_v, sg)

    def wait_gather(b):
        idx_v, _, rows_v, _, _, sg, _ = bufs[b]
        pltpu.make_async_copy(nn_hbm.at[idx_v], rows_v, sg).wait()

    def issue_scatter(b):
        _, dst_v, rows_v, _, _, _, ss = bufs[b]
        pltpu.async_copy(rows_v, agg_sp.at[dst_v], ss, add=True)

    def wait_scatter(b):
        _, dst_v, rows_v, _, _, _, ss = bufs[b]
        pltpu.make_async_copy(rows_v, agg_sp.at[dst_v], ss).wait()

    def mul(b):
        _, _, rows_v, h_v, _, _, _ = bufs[b]

        @plsc.parallel_loop(0, SC_B, unroll=8)
        def _(r):
            rows_v[r, pl.ds(0, 16)] = rows_v[r, pl.ds(0, 16)] * h_v[r, pl.ds(0, 16)]
            rows_v[r, pl.ds(16, 16)] = rows_v[r, pl.ds(16, 16)] * h_v[r, pl.ds(16, 16)]

    issue_loads(0, 0)
    wait_loads(0)
    issue_gather(0)
    issue_loads(1, 1)

    def body(k, _):
        for b, nb in ((0, 1), (1, 0)):
            t_next = 2 * k + b + 1

            @pl.when(t_next < SC_PT)
            def _():
                wait_loads(nb)
                issue_gather(nb)

            wait_gather(b)
            mul(b)
            issue_scatter(b)
            wait_scatter(b)
            t_pre = 2 * k + b + 2

            @pl.when(t_pre < SC_PT)
            def _():
                issue_loads(t_pre, b)

        return 0

    lax.fori_loop(0, SC_PT // 2, body, 0)
    plsc.subcore_barrier()
    pltpu.sync_copy(agg_sp.at[pl.ds(s * NT, NT)],
                    out_hbm.at[pl.ds(c * NPAD + s * NT, NT)])


@functools.partial(
    pl.kernel,
    out_type=jax.ShapeDtypeStruct((2 * NPAD, HALF), jnp.float32),
    mesh=plsc.VectorSubcoreMesh(core_axis_name="c", subcore_axis_name="s"),
    scratch_types=[
        pltpu.VMEM_SHARED((NPAD, HALF), jnp.float32),
        pltpu.VMEM((SC_B,), jnp.int32),
        pltpu.VMEM((SC_B,), jnp.int32),
        pltpu.VMEM((SC_B, HALF), jnp.float32),
        pltpu.VMEM((SC_B, HALF), jnp.float32),
        pltpu.VMEM((SC_B,), jnp.int32),
        pltpu.VMEM((SC_B,), jnp.int32),
        pltpu.VMEM((SC_B, HALF), jnp.float32),
        pltpu.VMEM((SC_B, HALF), jnp.float32),
        pltpu.SemaphoreType.DMA,
        pltpu.SemaphoreType.DMA,
        pltpu.SemaphoreType.DMA,
        pltpu.SemaphoreType.DMA,
        pltpu.SemaphoreType.DMA,
        pltpu.SemaphoreType.DMA,
    ],
    compiler_params=pltpu.CompilerParams(use_tc_tiling_on_sc=False),
)
def _segment_pass(src2_hbm, dst_hbm, nn_hbm, h_hbm, z_hbm, out_hbm, agg_sp,
                  idx0, dst0, rows0, h0, idx1, dst1, rows1, h1,
                  sl0, sg0, ss0, sl1, sg1, ss1):
    _seg_body(src2_hbm, dst_hbm, nn_hbm, h_hbm, z_hbm, out_hbm, agg_sp,
              idx0, dst0, rows0, h0, idx1, dst1, rows1, h1,
              sl0, sg0, ss0, sl1, sg1, ss1)


def kernel(node_type, edge_index, distance, node_index, source_index, target_index,
           select_edge_index, embedding, edge_mask, conv_params,
           W_nt1, b_nt1, W_nt2, b_nt2, W_et1, b_et1, W_et2, b_et2):
    N = node_type.shape[0]
    E = distance.shape[0]

    W1 = jnp.concatenate([p["W_cf1"] for p in conv_params], axis=1)
    W1 = jnp.pad(W1, ((0, KPAD - N_CENTERS), (0, 0)))
    B1 = jnp.concatenate([p["b_cf1"] for p in conv_params])[None, :]
    W2 = jax.scipy.linalg.block_diag(*[p["W_cf2"] for p in conv_params])
    B2 = jnp.concatenate([p["b_cf2"] for p in conv_params])[None, :]
    em_p = jnp.pad(edge_mask, (0, KPAD - N_CENTERS))[None, :]

    flag = jnp.zeros((E, 1), jnp.float32).at[select_edge_index].set(1.0)
    h_list = _compute_h_all(distance, flag, em_p, W1, B1, W2, B2)

    src = edge_index[0]
    dst = edge_index[1]
    src2 = jnp.concatenate([src, src + N])
    zeros_n = jnp.zeros((NPAD, HALF), jnp.float32)

    node = jnp.take(embedding, node_type, axis=0)
    for li, p in enumerate(conv_params):
        nn_cat = _new_node(node, p["W_nl1"]).reshape(2 * N, HALF)
        h_cat = h_list[li].reshape(2 * E, HALF)
        agg_cat = _segment_pass(src2, dst, nn_cat, h_cat, zeros_n)
        agg3d = agg_cat.reshape(2, NPAD, HALF)[:, :N, :]
        node = _update_node(node, agg3d, p["W_nl2"], p["b_nl2"], p["W_nl3"], p["b_nl3"])

    feature = node
    nsel = jnp.take(feature, node_index, axis=0)
    node_type_out = ((nsel @ W_nt1) + b_nt1) @ W_nt2 + b_nt2
    ef = jnp.concatenate([jnp.take(feature, source_index, axis=0),
                          jnp.take(feature, target_index, axis=0)], axis=1)
    edge_type_out = ((ef @ W_et1) + b_et1) @ W_et2 + b_et2
    return (node_type_out, edge_type_out)


# speedup vs baseline: 1.1025x; 1.1025x over previous
import functools

import jax
import jax.numpy as jnp
import numpy as np
from jax import lax
from jax.experimental import pallas as pl
from jax.experimental.pallas import tpu as pltpu
from jax.experimental.pallas import tpu_sc as plsc

DIM = 64
HALF = 32
CUTOFF = 5.0
N_CENTERS = 50
KPAD = 64
N_CONV = 3
HW = N_CONV * DIM
EBLK = 1000
NBLK = 10000

NNODE = 50000
NPAD = 50048
NEDGE = 800000
SC_B = 200
SC_PT = NEDGE // (16 * SC_B)
NT = NPAD // 16


_LOG2E = 1.4426950408889634
_LN2 = 0.6931471805599453


def _softplus(x, beta=0.5, threshold=14.0):
    y = beta * x
    sp = ((1.0 / beta) * _LN2) * jnp.log2(1.0 + jnp.exp2(_LOG2E * y))
    return jnp.where(y > threshold, x, sp)


def _h_body(d_ref, f_ref, em_ref, w1_ref, b1_ref, w2_ref, b2_ref, o0_ref):
    gap = CUTOFF / (N_CENTERS - 1)
    inv_gap = 1.0 / gap
    d = d_ref[...]
    ki = jax.lax.broadcasted_iota(jnp.int32, (1, KPAD), 1)
    centers = jnp.where(ki < N_CENTERS, ki.astype(jnp.float32) * gap, 0.0)
    rbf = jnp.exp(-inv_gap * (d - centers) ** 2)
    w1 = w1_ref[...]
    b1 = b1_ref[...]
    w2 = w2_ref[...]
    b2 = b2_ref[...]
    z = jnp.dot(rbf.astype(jnp.bfloat16), w1, preferred_element_type=jnp.float32) + b1
    h = jnp.dot(_softplus(z).astype(jnp.bfloat16), w2, preferred_element_type=jnp.float32) + b2
    zm = jnp.dot(em_ref[...].astype(jnp.bfloat16), w1, preferred_element_type=jnp.float32) + b1
    hm = jnp.dot(_softplus(zm).astype(jnp.bfloat16), w2, preferred_element_type=jnp.float32) + b2
    o0_ref[...] = jnp.where(f_ref[...] > 0.0, hm, h)


def _compute_h_all(distance, flag, edge_mask_p, W1, B1, W2, B2):
    E = distance.shape[0]
    grid = (E // EBLK,)

    return pl.pallas_call(
        _h_body,
        grid=grid,
        in_specs=[
            pl.BlockSpec((EBLK, 1), lambda i: (i, 0)),
            pl.BlockSpec((EBLK, 1), lambda i: (i, 0)),
            pl.BlockSpec((1, KPAD), lambda i: (0, 0)),
            pl.BlockSpec((KPAD, HW), lambda i: (0, 0)),
            pl.BlockSpec((1, HW), lambda i: (0, 0)),
            pl.BlockSpec((HW, HW), lambda i: (0, 0)),
            pl.BlockSpec((1, HW), lambda i: (0, 0)),
        ],
        out_specs=pl.BlockSpec((EBLK, HW), lambda i: (i, 0)),
        out_shape=jax.ShapeDtypeStruct((E, HW), jnp.float32),
    )(distance[:, None], flag, edge_mask_p,
      W1.astype(jnp.bfloat16), B1, W2.astype(jnp.bfloat16), B2)


def _nn_body(node_ref, w_ref, out_ref):
    nn = jnp.dot(node_ref[...], w_ref[...], preferred_element_type=jnp.float32)
    out_ref[...] = jnp.stack([nn[:, :HALF], nn[:, HALF:]], axis=0)


def _new_node(node, W_nl1):
    N = node.shape[0]
    return pl.pallas_call(
        _nn_body,
        grid=(N // NBLK,),
        in_specs=[
            pl.BlockSpec((NBLK, DIM), lambda i: (i, 0)),
            pl.BlockSpec((DIM, DIM), lambda i: (0, 0)),
        ],
        out_specs=pl.BlockSpec((2, NBLK, HALF), lambda i: (0, i, 0)),
        out_shape=jax.ShapeDtypeStruct((2, N, HALF), jnp.float32),
    )(node, W_nl1)


def _upd_body(node_ref, agg_ref, w2_ref, b2_ref, w3_ref, b3_ref, out_ref):
    agg = jnp.concatenate([agg_ref[0], agg_ref[1]], axis=1)
    cf1 = _softplus(jnp.dot(agg, w2_ref[...], preferred_element_type=jnp.float32) + b2_ref[...])
    out_ref[...] = node_ref[...] + (
        jnp.dot(cf1, w3_ref[...], preferred_element_type=jnp.float32) + b3_ref[...])


def _update_node(node, agg3d, W_nl2, b_nl2, W_nl3, b_nl3):
    N = node.shape[0]
    return pl.pallas_call(
        _upd_body,
        grid=(N // NBLK,),
        in_specs=[
            pl.BlockSpec((NBLK, DIM), lambda i: (i, 0)),
            pl.BlockSpec((2, NBLK, HALF), lambda i: (0, i, 0)),
            pl.BlockSpec((DIM, DIM), lambda i: (0, 0)),
            pl.BlockSpec((1, DIM), lambda i: (0, 0)),
            pl.BlockSpec((DIM, DIM), lambda i: (0, 0)),
            pl.BlockSpec((1, DIM), lambda i: (0, 0)),
        ],
        out_specs=pl.BlockSpec((NBLK, DIM), lambda i: (i, 0)),
        out_shape=jax.ShapeDtypeStruct((N, DIM), jnp.float32),
    )(node, agg3d, W_nl2, b_nl2[None, :], W_nl3, b_nl3[None, :])


def _seg_body(li, src2_hbm, dst_hbm, nn_hbm, h_hbm, z_hbm, out_hbm, agg_sp,
              idx0, dst0, rows0, h0, idx1, dst1, rows1, h1,
              sl0, sg0, ss0, sl1, sg1, ss1):
    c = lax.axis_index("c")
    s = lax.axis_index("s")
    pltpu.sync_copy(z_hbm.at[pl.ds(s * NT, NT)], agg_sp.at[pl.ds(s * NT, NT)])
    plsc.subcore_barrier()
    cE = c * NEDGE
    hcol = li * DIM + c * HALF
    base = s * (SC_PT * SC_B)
    bufs = ((idx0, dst0, rows0, h0, sl0, sg0, ss0),
            (idx1, dst1, rows1, h1, sl1, sg1, ss1))

    def issue_loads(t, b):
        idx_v, dst_v, _, h_v, sl, _, _ = bufs[b]
        off = base + t * SC_B
        pltpu.async_copy(src2_hbm.at[pl.ds(cE + off, SC_B)], idx_v, sl)
        pltpu.async_copy(dst_hbm.at[pl.ds(off, SC_B)], dst_v, sl)
        pltpu.async_copy(h_hbm.at[pl.ds(off, SC_B), pl.ds(hcol, HALF)], h_v, sl)

    def wait_loads(b):
        idx_v, dst_v, _, h_v, sl, _, _ = bufs[b]
        pltpu.make_async_copy(src2_hbm.at[pl.ds(0, SC_B)], idx_v, sl).wait()
        pltpu.make_async_copy(dst_hbm.at[pl.ds(0, SC_B)], dst_v, sl).wait()
        pltpu.make_async_copy(h_hbm.at[pl.ds(0, SC_B), pl.ds(hcol, HALF)], h_v, sl).wait()

    def issue_gather(b):
        idx_v, _, rows_v, _, _, sg, _ = bufs[b]
        pltpu.async_copy(nn_hbm.at[idx_v], rows_v, sg)

    def wait_gather(b):
        idx_v, _, rows_v, _, _, sg, _ = bufs[b]
        pltpu.make_async_copy(nn_hbm.at[idx_v], rows_v, sg).wait()

    def issue_scatter(b):
        _, dst_v, rows_v, _, _, _, ss = bufs[b]
        pltpu.async_copy(rows_v, agg_sp.at[dst_v], ss, add=True)

    def wait_scatter(b):
        _, dst_v, rows_v, _, _, _, ss = bufs[b]
        pltpu.make_async_copy(rows_v, agg_sp.at[dst_v], ss).wait()

    def mul(b):
        _, _, rows_v, h_v, _, _, _ = bufs[b]

        @plsc.parallel_loop(0, SC_B, unroll=8)
        def _(r):
            rows_v[r, pl.ds(0, 16)] = rows_v[r, pl.ds(0, 16)] * h_v[r, pl.ds(0, 16)]
            rows_v[r, pl.ds(16, 16)] = rows_v[r, pl.ds(16, 16)] * h_v[r, pl.ds(16, 16)]

    issue_loads(0, 0)
    wait_loads(0)
    issue_gather(0)
    issue_loads(1, 1)

    def body(k, _):
        for b, nb in ((0, 1), (1, 0)):
            t_next = 2 * k + b + 1

            @pl.when(t_next < SC_PT)
            def _():
                wait_loads(nb)
                issue_gather(nb)

            wait_gather(b)
            mul(b)
            issue_scatter(b)
            wait_scatter(b)
            t_pre = 2 * k + b + 2

            @pl.when(t_pre < SC_PT)
            def _():
                issue_loads(t_pre, b)

        return 0

    lax.fori_loop(0, SC_PT // 2, body, 0)
    plsc.subcore_barrier()
    pltpu.sync_copy(agg_sp.at[pl.ds(s * NT, NT)],
                    out_hbm.at[pl.ds(c * NPAD + s * NT, NT)])


def _make_segment_pass(li):
    @functools.partial(
        pl.kernel,
        out_type=jax.ShapeDtypeStruct((2 * NPAD, HALF), jnp.float32),
        mesh=plsc.VectorSubcoreMesh(core_axis_name="c", subcore_axis_name="s"),
        scratch_types=[
            pltpu.VMEM_SHARED((NPAD, HALF), jnp.float32),
            pltpu.VMEM((SC_B,), jnp.int32),
            pltpu.VMEM((SC_B,), jnp.int32),
            pltpu.VMEM((SC_B, HALF), jnp.float32),
            pltpu.VMEM((SC_B, HALF), jnp.float32),
            pltpu.VMEM((SC_B,), jnp.int32),
            pltpu.VMEM((SC_B,), jnp.int32),
            pltpu.VMEM((SC_B, HALF), jnp.float32),
            pltpu.VMEM((SC_B, HALF), jnp.float32),
            pltpu.SemaphoreType.DMA,
            pltpu.SemaphoreType.DMA,
            pltpu.SemaphoreType.DMA,
            pltpu.SemaphoreType.DMA,
            pltpu.SemaphoreType.DMA,
            pltpu.SemaphoreType.DMA,
        ],
        compiler_params=pltpu.CompilerParams(use_tc_tiling_on_sc=False),
        name=f"segment_pass_l{li}",
    )
    def _segment_pass_li(src2_hbm, dst_hbm, nn_hbm, h_hbm, z_hbm, out_hbm, agg_sp,
                         idx0, dst0, rows0, h0, idx1, dst1, rows1, h1,
                         sl0, sg0, ss0, sl1, sg1, ss1):
        _seg_body(li, src2_hbm, dst_hbm, nn_hbm, h_hbm, z_hbm, out_hbm, agg_sp,
                  idx0, dst0, rows0, h0, idx1, dst1, rows1, h1,
                  sl0, sg0, ss0, sl1, sg1, ss1)

    return _segment_pass_li


_SEGMENT_PASS = tuple(_make_segment_pass(li) for li in range(N_CONV))


def kernel(node_type, edge_index, distance, node_index, source_index, target_index,
           select_edge_index, embedding, edge_mask, conv_params,
           W_nt1, b_nt1, W_nt2, b_nt2, W_et1, b_et1, W_et2, b_et2):
    N = node_type.shape[0]
    E = distance.shape[0]

    W1 = jnp.concatenate([p["W_cf1"] for p in conv_params], axis=1)
    W1 = jnp.pad(W1, ((0, KPAD - N_CENTERS), (0, 0)))
    B1 = jnp.concatenate([p["b_cf1"] for p in conv_params])[None, :]
    W2 = jax.scipy.linalg.block_diag(*[p["W_cf2"] for p in conv_params])
    B2 = jnp.concatenate([p["b_cf2"] for p in conv_params])[None, :]
    em_p = jnp.pad(edge_mask, (0, KPAD - N_CENTERS))[None, :]

    flag = jnp.zeros((E, 1), jnp.float32).at[select_edge_index].set(1.0)
    h_all = _compute_h_all(distance, flag, em_p, W1, B1, W2, B2)

    src = edge_index[0]
    dst = edge_index[1]
    src2 = jnp.concatenate([src, src + N])
    zeros_n = jnp.zeros((NPAD, HALF), jnp.float32)

    node = jnp.take(embedding, node_type, axis=0)
    for li, p in enumerate(conv_params):
        nn_cat = _new_node(node, p["W_nl1"]).reshape(2 * N, HALF)
        agg_cat = _SEGMENT_PASS[li](src2, dst, nn_cat, h_all, zeros_n)
        agg3d = agg_cat.reshape(2, NPAD, HALF)[:, :N, :]
        node = _update_node(node, agg3d, p["W_nl2"], p["b_nl2"], p["W_nl3"], p["b_nl3"])

    feature = node
    nsel = jnp.take(feature, node_index, axis=0)
    node_type_out = ((nsel @ W_nt1) + b_nt1) @ W_nt2 + b_nt2
    ef = jnp.concatenate([jnp.take(feature, source_index, axis=0),
                          jnp.take(feature, target_index, axis=0)], axis=1)
    edge_type_out = ((ef @ W_et1) + b_et1) @ W_et2 + b_et2
    return (node_type_out, edge_type_out)
